# baseline (speedup 1.0000x reference)
import functools

import jax
import jax.numpy as jnp
from jax import lax
from jax.experimental import pallas as pl
from jax.experimental.pallas import tpu as pltpu
from jax.experimental.pallas import tpu_sc as plsc

D = 64
CHUNK = 128

_info = plsc.get_sparse_core_info()
_NC, _NS = _info.num_cores, _info.num_subcores
_NW = _NC * _NS


@functools.lru_cache(maxsize=None)
def _make_gather(B: int):
    assert B % (_NW * CHUNK) == 0
    cpw = B // (_NW * CHUNK)
    assert cpw % 2 == 0
    mesh = plsc.VectorSubcoreMesh(core_axis_name="c", subcore_axis_name="s")

    @functools.partial(
        pl.kernel,
        mesh=mesh,
        compiler_params=pltpu.CompilerParams(use_tc_tiling_on_sc=False),
        out_type=jax.ShapeDtypeStruct((B, D), jnp.float32),
        scratch_types=[
            pltpu.VMEM((cpw, CHUNK), jnp.int32),
            pltpu.VMEM((CHUNK, D), jnp.float32),
            pltpu.VMEM((CHUNK, D), jnp.float32),
            pltpu.SemaphoreType.DMA,
            pltpu.SemaphoreType.DMA,
        ],
    )
    def gather_kernel(idx_hbm, table_hbm, out_hbm, idx_v, buf0, buf1,
                      sem0, sem1):
        wid = lax.axis_index("s") * _NC + lax.axis_index("c")
        chunk0 = wid * cpw
        row0 = chunk0 * CHUNK

        pltpu.sync_copy(idx_hbm.at[pl.ds(chunk0, cpw)], idx_v)

        def start_gather(j, buf, sem):
            pltpu.async_copy(table_hbm.at[idx_v.at[j]], buf, sem)

        def wait_gather(buf, sem):
            pltpu.make_async_copy(table_hbm.at[idx_v.at[0]], buf, sem).wait()

        def write_out(j, buf):
            pltpu.sync_copy(buf, out_hbm.at[pl.ds(row0 + j * CHUNK, CHUNK)])

        start_gather(0, buf0, sem0)

        def body(i, carry):
            j0 = 2 * i
            start_gather(j0 + 1, buf1, sem1)
            wait_gather(buf0, sem0)
            write_out(j0, buf0)

            @pl.when(j0 + 2 < cpw)
            def _():
                start_gather(j0 + 2, buf0, sem0)

            wait_gather(buf1, sem1)
            write_out(j0 + 1, buf1)
            return carry

        lax.fori_loop(0, cpw // 2, body, 0)

    return gather_kernel


def kernel(inputs, table):
    B, H = inputs.shape
    n = B * H
    idx2d = inputs.reshape(n // CHUNK, CHUNK)
    out = _make_gather(n)(idx2d, table)
    return out.reshape(B, H, D)
